# baseline (speedup 1.0000x reference)
import functools

import jax
import jax.numpy as jnp
from jax import lax
from jax.experimental import pallas as pl
from jax.experimental.pallas import tpu as pltpu
from jax.experimental.pallas import tpu_sc as plsc

_N = 10000
_E = 320000
_D = 128
_H = 64
_SB = 512
_NT = 16
_NP = 10240
_RPT = _NP // _NT
_EP = 327680
_NSB = _EP // _SB
_ROWBLK = 1000


def _sc_mesh():
    return plsc.VectorSubcoreMesh(core_axis_name="c", subcore_axis_name="s")


def _zero_buf(buf, nrows):
    zero16 = jnp.zeros((16,), jnp.float32)

    def zrow(r, carry):
        for j in range(_H // 16):
            buf[r, pl.ds(j * 16, 16)] = zero16
        return carry

    lax.fori_loop(0, nrows, zrow, 0)


def _sc_scratch():
    bufs = []
    for _ in range(2):
        bufs += [pltpu.VMEM((_SB,), jnp.int32),
                 pltpu.VMEM((_SB,), jnp.int32),
                 pltpu.VMEM((_SB,), jnp.float32),
                 pltpu.VMEM((_SB, _H), jnp.float32)]
    return bufs + [
        pltpu.VMEM_SHARED((_NP, _H), jnp.float32),
        pltpu.SemaphoreType.DMA,
        pltpu.SemaphoreType.DMA,
        pltpu.SemaphoreType.DMA,
        pltpu.SemaphoreType.DMA,
        pltpu.SemaphoreType.DMA,
        pltpu.SemaphoreType.DMA,
    ]


def _pipeline(sb0, nsb, bufs, acc_sh, t_h, src_h, dst_h, w_h):
    def load_idx(hb, sbi):
        src_v, dst_v, w_v, sem = hb[0], hb[1], hb[2], hb[4]
        e0 = (sb0 + sbi) * _SB
        pltpu.async_copy(src_h.at[pl.ds(e0, _SB)], src_v, sem)
        pltpu.async_copy(dst_h.at[pl.ds(e0, _SB)], dst_v, sem)
        pltpu.async_copy(w_h.at[pl.ds(e0, _SB)], w_v, sem)

    def wait_idx(hb):
        src_v, dst_v, w_v, sem = hb[0], hb[1], hb[2], hb[4]
        e0 = sb0 * _SB
        pltpu.make_async_copy(src_h.at[pl.ds(e0, _SB)], src_v, sem).wait()
        pltpu.make_async_copy(dst_h.at[pl.ds(e0, _SB)], dst_v, sem).wait()
        pltpu.make_async_copy(w_h.at[pl.ds(e0, _SB)], w_v, sem).wait()

    def fire_gather(hb):
        src_v, rows_v, sem = hb[0], hb[3], hb[5]
        pltpu.async_copy(t_h.at[src_v], rows_v, sem)

    def drain_gather(hb):
        src_v, rows_v, sem = hb[0], hb[3], hb[5]
        pltpu.make_async_copy(t_h.at[src_v], rows_v, sem).wait()

    def scale(hb):
        w_v, rows_v = hb[2], hb[3]

        def grp(g, carry):
            wv16 = w_v[pl.ds(g * 16, 16)]
            for i in range(16):
                r = g * 16 + i
                wv = wv16[i]
                for q in range(_H // 16):
                    sl = pl.ds(q * 16, 16)
                    rows_v[r, sl] = rows_v[r, sl] * wv
            return carry

        lax.fori_loop(0, _SB // 16, grp, 0)

    def fire_scatter(hb):
        dst_v, rows_v, sem = hb[1], hb[3], hb[6]
        pltpu.async_copy(rows_v, acc_sh.at[dst_v], sem, add=True)

    def drain_scatter(hb):
        dst_v, rows_v, sem = hb[1], hb[3], hb[6]
        pltpu.make_async_copy(rows_v, acc_sh.at[dst_v], sem).wait()

    buf_a, buf_b = bufs
    npairs = nsb // 2

    load_idx(buf_a, 0)
    load_idx(buf_b, 1)
    wait_idx(buf_a)
    fire_gather(buf_a)
    wait_idx(buf_b)
    fire_gather(buf_b)

    def half(hb, sb_next, is_not_last):
        drain_gather(hb)
        scale(hb)
        fire_scatter(hb)
        drain_scatter(hb)

        @pl.when(is_not_last)
        def _():
            load_idx(hb, sb_next)
            wait_idx(hb)
            fire_gather(hb)

    def pair(pi, carry):
        not_last = pi < npairs - 1
        half(buf_a, pi * 2 + 2, not_last)
        half(buf_b, pi * 2 + 3, not_last)
        return carry

    lax.fori_loop(0, npairs, pair, 0)


def _zero_acc(rows_a, acc_sh, s):
    _zero_buf(rows_a, _SB)
    pltpu.sync_copy(rows_a, acc_sh.at[pl.ds(s * _RPT, _SB)])
    pltpu.sync_copy(rows_a.at[pl.ds(0, _RPT - _SB)],
                    acc_sh.at[pl.ds(s * _RPT + _SB, _RPT - _SB)])


def _write_out(rows_a, acc_sh, o_slice, s):
    r0 = s * _RPT
    pltpu.sync_copy(acc_sh.at[pl.ds(r0, _SB)], rows_a)
    pltpu.sync_copy(rows_a, o_slice.at[pl.ds(r0, _SB)])
    pltpu.sync_copy(acc_sh.at[pl.ds(r0 + _SB, _RPT - _SB)],
                    rows_a.at[pl.ds(0, _RPT - _SB)])
    pltpu.sync_copy(rows_a.at[pl.ds(0, _RPT - _SB)],
                    o_slice.at[pl.ds(r0 + _SB, _RPT - _SB)])


def _spmm_hop1(src1, dst1, w1, t0, t1):
    spt = _NSB // _NT

    @functools.partial(
        pl.kernel,
        mesh=_sc_mesh(),
        out_type=[jax.ShapeDtypeStruct((_NP, _H), jnp.float32),
                  jax.ShapeDtypeStruct((_NP, _H), jnp.float32)],
        scratch_types=_sc_scratch(),
        compiler_params=pltpu.CompilerParams(use_tc_tiling_on_sc=False),
    )
    def k(src_h, dst_h, w_h, t0_h, t1_h, o0_h, o1_h,
          src_a, dst_a, w_a, rows_a, src_b, dst_b, w_b, rows_b, acc_sh,
          sida, sidb, sga, sgb, ssa, ssb):
        c = lax.axis_index("c")
        s = lax.axis_index("s")
        buf_a = (src_a, dst_a, w_a, rows_a, sida, sga, ssa)
        buf_b = (src_b, dst_b, w_b, rows_b, sidb, sgb, ssb)
        _zero_acc(rows_a, acc_sh, s)
        plsc.subcore_barrier()

        sb0 = s * spt

        @pl.when(c == 0)
        def _():
            _pipeline(sb0, spt, (buf_a, buf_b), acc_sh, t0_h,
                      src_h, dst_h, w_h)

        @pl.when(c == 1)
        def _():
            _pipeline(sb0, spt, (buf_a, buf_b), acc_sh, t1_h,
                      src_h, dst_h, w_h)

        plsc.subcore_barrier()

        @pl.when(c == 0)
        def _():
            _write_out(rows_a, acc_sh, o0_h, s)

        @pl.when(c == 1)
        def _():
            _write_out(rows_a, acc_sh, o1_h, s)

    return k(src1, dst1, w1, t0, t1)


def _spmm_hop2(src1, dst1, w1, t):
    half_sb = _NSB // 2
    spt = half_sb // _NT

    @functools.partial(
        pl.kernel,
        mesh=_sc_mesh(),
        out_type=jax.ShapeDtypeStruct((2, _NP, _H), jnp.float32),
        scratch_types=_sc_scratch(),
        compiler_params=pltpu.CompilerParams(use_tc_tiling_on_sc=False),
    )
    def k(src_h, dst_h, w_h, t_h, o_h,
          src_a, dst_a, w_a, rows_a, src_b, dst_b, w_b, rows_b, acc_sh,
          sida, sidb, sga, sgb, ssa, ssb):
        c = lax.axis_index("c")
        s = lax.axis_index("s")
        buf_a = (src_a, dst_a, w_a, rows_a, sida, sga, ssa)
        buf_b = (src_b, dst_b, w_b, rows_b, sidb, sgb, ssb)
        _zero_acc(rows_a, acc_sh, s)
        plsc.subcore_barrier()

        sb0 = c * half_sb + s * spt
        _pipeline(sb0, spt, (buf_a, buf_b), acc_sh, t_h,
                  src_h, dst_h, w_h)

        plsc.subcore_barrier()
        _write_out(rows_a, acc_sh, o_h.at[c], s)

    return k(src1, dst1, w1, t)


def _tc_first(x, wcat):
    def body(x_ref, w_ref, o0_ref, o1_ref):
        u = jnp.dot(x_ref[...], w_ref[...],
                    preferred_element_type=jnp.float32)
        o0_ref[...] = u[:, :_H]
        o1_ref[...] = u[:, _H:]

    return pl.pallas_call(
        body,
        grid=(_N // _ROWBLK,),
        in_specs=[pl.BlockSpec((_ROWBLK, _D), lambda i: (i, 0)),
                  pl.BlockSpec((_D, _D), lambda i: (0, 0))],
        out_specs=[pl.BlockSpec((_ROWBLK, _H), lambda i: (i, 0)),
                   pl.BlockSpec((_ROWBLK, _H), lambda i: (i, 0))],
        out_shape=[jax.ShapeDtypeStruct((_N, _H), jnp.float32),
                   jax.ShapeDtypeStruct((_N, _H), jnp.float32)],
    )(x, wcat)


def _tc_layer(keep, p0, p1, wcat, bvec):
    def body(k_ref, p0_ref, p1_ref, w_ref, b_ref, o0_ref, o1_ref):
        wl = w_ref[...]
        upper = p0_ref[...] + p1_ref[...]
        u = (jnp.dot(k_ref[...], wl[:_H, :],
                     preferred_element_type=jnp.float32)
             + jnp.dot(upper, wl[_H:, :],
                       preferred_element_type=jnp.float32)
             + jnp.dot(b_ref[...], wl, preferred_element_type=jnp.float32))
        o0_ref[...] = u[:, :_H]
        o1_ref[...] = u[:, _H:]

    return pl.pallas_call(
        body,
        grid=(_N // _ROWBLK,),
        in_specs=[pl.BlockSpec((_ROWBLK, _H), lambda i: (i, 0)),
                  pl.BlockSpec((_ROWBLK, _H), lambda i: (i, 0)),
                  pl.BlockSpec((_ROWBLK, _H), lambda i: (i, 0)),
                  pl.BlockSpec((_D, _D), lambda i: (0, 0)),
                  pl.BlockSpec((1, _D), lambda i: (0, 0))],
        out_specs=[pl.BlockSpec((_ROWBLK, _H), lambda i: (i, 0)),
                   pl.BlockSpec((_ROWBLK, _H), lambda i: (i, 0))],
        out_shape=[jax.ShapeDtypeStruct((_N, _H), jnp.float32),
                   jax.ShapeDtypeStruct((_N, _H), jnp.float32)],
    )(keep, p0, p1, wcat, bvec)


def _tc_final(keep, p0, p1, wcp, bvec, bcp):
    def body(k_ref, p0_ref, p1_ref, w_ref, b_ref, bc_ref, o_ref):
        wl = w_ref[...]
        upper = p0_ref[...] + p1_ref[...]
        o_ref[...] = (jnp.dot(k_ref[...], wl[:_H, :],
                              preferred_element_type=jnp.float32)
                      + jnp.dot(upper, wl[_H:, :],
                                preferred_element_type=jnp.float32)
                      + jnp.dot(b_ref[...], wl,
                                preferred_element_type=jnp.float32)
                      + bc_ref[...])

    return pl.pallas_call(
        body,
        grid=(_N // _ROWBLK,),
        in_specs=[pl.BlockSpec((_ROWBLK, _H), lambda i: (i, 0)),
                  pl.BlockSpec((_ROWBLK, _H), lambda i: (i, 0)),
                  pl.BlockSpec((_ROWBLK, _H), lambda i: (i, 0)),
                  pl.BlockSpec((_D, _D), lambda i: (0, 0)),
                  pl.BlockSpec((1, _D), lambda i: (0, 0)),
                  pl.BlockSpec((1, _D), lambda i: (0, 0))],
        out_specs=pl.BlockSpec((_ROWBLK, _D), lambda i: (i, 0)),
        out_shape=jax.ShapeDtypeStruct((_N, _D), jnp.float32),
    )(keep, p0, p1, wcp, bvec, bcp)


def kernel(x, edge_index, edge_weight, W, b, Wc, bc):
    pad = _EP - _E
    src1 = jnp.concatenate([edge_index[0], jnp.zeros((pad,), jnp.int32)])
    dst1 = jnp.concatenate([edge_index[1], jnp.zeros((pad,), jnp.int32)])
    w1 = jnp.concatenate([edge_weight, jnp.zeros((pad,), jnp.float32)])
    nclass = Wc.shape[1]

    t0, t1 = _tc_first(x, jnp.concatenate([W[0, 0], W[0, 1]], axis=1))
    for l in range(W.shape[0]):
        keep, upper = _spmm_hop1(src1, dst1, w1, t0, t1)
        parts = _spmm_hop2(src1, dst1, w1, upper)
        p0, p1 = parts[0], parts[1]
        bvec = jnp.concatenate([b[l, 0], b[l, 1]])[None, :]
        if l + 1 < W.shape[0]:
            wcat = jnp.concatenate([W[l + 1, 0], W[l + 1, 1]], axis=1)
            t0, t1 = _tc_layer(keep, p0, p1, wcat, bvec)
        else:
            wcp = jnp.pad(Wc, ((0, 0), (0, _D - nclass)))
            bcp = jnp.pad(bc, (0, _D - nclass))[None, :]
            out = _tc_final(keep, p0, p1, wcp, bvec, bcp)
            return out[:, :nclass]


# speedup vs baseline: 1.0789x; 1.0789x over previous
import functools

import jax
import jax.numpy as jnp
from jax import lax
from jax.experimental import pallas as pl
from jax.experimental.pallas import tpu as pltpu
from jax.experimental.pallas import tpu_sc as plsc

_N = 10000
_E = 320000
_D = 128
_H = 64
_SB = 256
_NT = 16
_NP = 10240
_RPT = _NP // _NT
_EP = 327680
_NSB = _EP // _SB
_ROWBLK = 640


def _sc_mesh():
    return plsc.VectorSubcoreMesh(core_axis_name="c", subcore_axis_name="s")


def _sc_scratch():
    bufs = []
    for _ in range(2):
        bufs += [pltpu.VMEM((_SB,), jnp.int32),
                 pltpu.VMEM((_SB,), jnp.int32),
                 pltpu.VMEM((_SB,), jnp.float32),
                 pltpu.VMEM((_SB, _H), jnp.float32)]
    return bufs + [
        pltpu.VMEM_SHARED((_NP, _H), jnp.float32),
        pltpu.VMEM_SHARED((_NP, _H), jnp.float32),
        pltpu.SemaphoreType.DMA,
        pltpu.SemaphoreType.DMA,
        pltpu.SemaphoreType.DMA,
        pltpu.SemaphoreType.DMA,
        pltpu.SemaphoreType.DMA,
        pltpu.SemaphoreType.DMA,
        pltpu.SemaphoreType.DMA,
    ]


def _stage_and_zero(t_h, tbl_sh, acc_sh, rows_a, s, sstage):
    r0 = s * _RPT
    cp = pltpu.async_copy(t_h.at[pl.ds(r0, _RPT)],
                          tbl_sh.at[pl.ds(r0, _RPT)], sstage)
    zero16 = jnp.zeros((16,), jnp.float32)

    def zrow(r, carry):
        for j in range(_H // 16):
            rows_a[r, pl.ds(j * 16, 16)] = zero16
        return carry

    lax.fori_loop(0, _SB, zrow, 0)
    for off, ln in _stripe_pieces():
        pltpu.sync_copy(rows_a.at[pl.ds(0, ln)],
                        acc_sh.at[pl.ds(r0 + off, ln)])
    cp.wait()


def _stripe_pieces():
    pieces, off = [], 0
    while off < _RPT:
        ln = min(_SB, _RPT - off)
        pieces.append((off, ln))
        off += ln
    return pieces


def _write_out(rows_a, acc_sh, o_slice, s):
    r0 = s * _RPT
    for off, ln in _stripe_pieces():
        pltpu.sync_copy(acc_sh.at[pl.ds(r0 + off, ln)],
                        rows_a.at[pl.ds(0, ln)])
        pltpu.sync_copy(rows_a.at[pl.ds(0, ln)],
                        o_slice.at[pl.ds(r0 + off, ln)])


def _pipeline(sb0, nsb, bufs, tbl_sh, acc_sh, src_h, dst_h, w_h):
    def load_idx(hb, sbi):
        src_v, dst_v, w_v, sem = hb[0], hb[1], hb[2], hb[4]
        e0 = (sb0 + sbi) * _SB
        pltpu.async_copy(src_h.at[pl.ds(e0, _SB)], src_v, sem)
        pltpu.async_copy(dst_h.at[pl.ds(e0, _SB)], dst_v, sem)
        pltpu.async_copy(w_h.at[pl.ds(e0, _SB)], w_v, sem)

    def wait_idx(hb):
        src_v, dst_v, w_v, sem = hb[0], hb[1], hb[2], hb[4]
        e0 = sb0 * _SB
        pltpu.make_async_copy(src_h.at[pl.ds(e0, _SB)], src_v, sem).wait()
        pltpu.make_async_copy(dst_h.at[pl.ds(e0, _SB)], dst_v, sem).wait()
        pltpu.make_async_copy(w_h.at[pl.ds(e0, _SB)], w_v, sem).wait()

    def fire_gather(hb):
        src_v, rows_v, sem = hb[0], hb[3], hb[5]
        pltpu.async_copy(tbl_sh.at[src_v], rows_v, sem)

    def drain_gather(hb):
        src_v, rows_v, sem = hb[0], hb[3], hb[5]
        pltpu.make_async_copy(tbl_sh.at[src_v], rows_v, sem).wait()

    def scale(hb):
        w_v, rows_v = hb[2], hb[3]

        def grp(g, carry):
            wv16 = w_v[pl.ds(g * 16, 16)]
            for i in range(16):
                r = g * 16 + i
                wv = wv16[i]
                for q in range(_H // 16):
                    sl = pl.ds(q * 16, 16)
                    rows_v[r, sl] = rows_v[r, sl] * wv
            return carry

        lax.fori_loop(0, _SB // 16, grp, 0)

    def fire_scatter(hb):
        dst_v, rows_v, sem = hb[1], hb[3], hb[6]
        pltpu.async_copy(rows_v, acc_sh.at[dst_v], sem, add=True)

    def drain_scatter(hb):
        dst_v, rows_v, sem = hb[1], hb[3], hb[6]
        pltpu.make_async_copy(rows_v, acc_sh.at[dst_v], sem).wait()

    buf_a, buf_b = bufs
    npairs = nsb // 2

    load_idx(buf_a, 0)
    load_idx(buf_b, 1)
    wait_idx(buf_a)
    fire_gather(buf_a)
    wait_idx(buf_b)
    fire_gather(buf_b)

    def half(hb, sb_next, is_not_last):
        drain_gather(hb)
        scale(hb)
        fire_scatter(hb)
        drain_scatter(hb)

        @pl.when(is_not_last)
        def _():
            load_idx(hb, sb_next)
            wait_idx(hb)
            fire_gather(hb)

    def pair(pi, carry):
        not_last = pi < npairs - 1
        half(buf_a, pi * 2 + 2, not_last)
        half(buf_b, pi * 2 + 3, not_last)
        return carry

    lax.fori_loop(0, npairs, pair, 0)


def _spmm_hop1(src1, dst1, w1, t0, t1):
    spt = _NSB // _NT

    @functools.partial(
        pl.kernel,
        mesh=_sc_mesh(),
        out_type=[jax.ShapeDtypeStruct((_NP, _H), jnp.float32),
                  jax.ShapeDtypeStruct((_NP, _H), jnp.float32)],
        scratch_types=_sc_scratch(),
        compiler_params=pltpu.CompilerParams(use_tc_tiling_on_sc=False),
    )
    def k(src_h, dst_h, w_h, t0_h, t1_h, o0_h, o1_h,
          src_a, dst_a, w_a, rows_a, src_b, dst_b, w_b, rows_b,
          tbl_sh, acc_sh, sida, sidb, sga, sgb, ssa, ssb, sstage):
        c = lax.axis_index("c")
        s = lax.axis_index("s")
        buf_a = (src_a, dst_a, w_a, rows_a, sida, sga, ssa)
        buf_b = (src_b, dst_b, w_b, rows_b, sidb, sgb, ssb)

        @pl.when(c == 0)
        def _():
            _stage_and_zero(t0_h, tbl_sh, acc_sh, rows_a, s, sstage)

        @pl.when(c == 1)
        def _():
            _stage_and_zero(t1_h, tbl_sh, acc_sh, rows_a, s, sstage)

        plsc.subcore_barrier()

        sb0 = s * spt
        _pipeline(sb0, spt, (buf_a, buf_b), tbl_sh, acc_sh,
                  src_h, dst_h, w_h)

        plsc.subcore_barrier()

        @pl.when(c == 0)
        def _():
            _write_out(rows_a, acc_sh, o0_h, s)

        @pl.when(c == 1)
        def _():
            _write_out(rows_a, acc_sh, o1_h, s)

    return k(src1, dst1, w1, t0, t1)


def _spmm_hop2(src1, dst1, w1, t):
    half_sb = _NSB // 2
    spt = half_sb // _NT

    @functools.partial(
        pl.kernel,
        mesh=_sc_mesh(),
        out_type=jax.ShapeDtypeStruct((2, _NP, _H), jnp.float32),
        scratch_types=_sc_scratch(),
        compiler_params=pltpu.CompilerParams(use_tc_tiling_on_sc=False),
    )
    def k(src_h, dst_h, w_h, t_h, o_h,
          src_a, dst_a, w_a, rows_a, src_b, dst_b, w_b, rows_b,
          tbl_sh, acc_sh, sida, sidb, sga, sgb, ssa, ssb, sstage):
        c = lax.axis_index("c")
        s = lax.axis_index("s")
        buf_a = (src_a, dst_a, w_a, rows_a, sida, sga, ssa)
        buf_b = (src_b, dst_b, w_b, rows_b, sidb, sgb, ssb)

        _stage_and_zero(t_h, tbl_sh, acc_sh, rows_a, s, sstage)
        plsc.subcore_barrier()

        sb0 = c * half_sb + s * spt
        _pipeline(sb0, spt, (buf_a, buf_b), tbl_sh, acc_sh,
                  src_h, dst_h, w_h)

        plsc.subcore_barrier()
        _write_out(rows_a, acc_sh, o_h.at[c], s)

    return k(src1, dst1, w1, t)


def _tc_first(x, wcat):
    def body(x_ref, w_ref, o0_ref, o1_ref):
        u = jnp.dot(x_ref[...], w_ref[...],
                    preferred_element_type=jnp.float32)
        o0_ref[...] = u[:, :_H]
        o1_ref[...] = u[:, _H:]

    return pl.pallas_call(
        body,
        grid=(_NP // _ROWBLK,),
        in_specs=[pl.BlockSpec((_ROWBLK, _D), lambda i: (i, 0)),
                  pl.BlockSpec((_D, _D), lambda i: (0, 0))],
        out_specs=[pl.BlockSpec((_ROWBLK, _H), lambda i: (i, 0)),
                   pl.BlockSpec((_ROWBLK, _H), lambda i: (i, 0))],
        out_shape=[jax.ShapeDtypeStruct((_NP, _H), jnp.float32),
                   jax.ShapeDtypeStruct((_NP, _H), jnp.float32)],
    )(x, wcat)


def _tc_layer(keep, p0, p1, wcat, bvec):
    def body(k_ref, p0_ref, p1_ref, w_ref, b_ref, o0_ref, o1_ref):
        wl = w_ref[...]
        upper = p0_ref[...] + p1_ref[...]
        u = (jnp.dot(k_ref[...], wl[:_H, :],
                     preferred_element_type=jnp.float32)
             + jnp.dot(upper, wl[_H:, :],
                       preferred_element_type=jnp.float32)
             + jnp.dot(b_ref[...], wl, preferred_element_type=jnp.float32))
        o0_ref[...] = u[:, :_H]
        o1_ref[...] = u[:, _H:]

    return pl.pallas_call(
        body,
        grid=(_NP // _ROWBLK,),
        in_specs=[pl.BlockSpec((_ROWBLK, _H), lambda i: (i, 0)),
                  pl.BlockSpec((_ROWBLK, _H), lambda i: (i, 0)),
                  pl.BlockSpec((_ROWBLK, _H), lambda i: (i, 0)),
                  pl.BlockSpec((_D, _D), lambda i: (0, 0)),
                  pl.BlockSpec((1, _D), lambda i: (0, 0))],
        out_specs=[pl.BlockSpec((_ROWBLK, _H), lambda i: (i, 0)),
                   pl.BlockSpec((_ROWBLK, _H), lambda i: (i, 0))],
        out_shape=[jax.ShapeDtypeStruct((_NP, _H), jnp.float32),
                   jax.ShapeDtypeStruct((_NP, _H), jnp.float32)],
    )(keep, p0, p1, wcat, bvec)


def _tc_final(keep, p0, p1, wcp, bvec, bcp):
    def body(k_ref, p0_ref, p1_ref, w_ref, b_ref, bc_ref, o_ref):
        wl = w_ref[...]
        upper = p0_ref[...] + p1_ref[...]
        o_ref[...] = (jnp.dot(k_ref[...], wl[:_H, :],
                              preferred_element_type=jnp.float32)
                      + jnp.dot(upper, wl[_H:, :],
                                preferred_element_type=jnp.float32)
                      + jnp.dot(b_ref[...], wl,
                                preferred_element_type=jnp.float32)
                      + bc_ref[...])

    return pl.pallas_call(
        body,
        grid=(_NP // _ROWBLK,),
        in_specs=[pl.BlockSpec((_ROWBLK, _H), lambda i: (i, 0)),
                  pl.BlockSpec((_ROWBLK, _H), lambda i: (i, 0)),
                  pl.BlockSpec((_ROWBLK, _H), lambda i: (i, 0)),
                  pl.BlockSpec((_D, _D), lambda i: (0, 0)),
                  pl.BlockSpec((1, _D), lambda i: (0, 0)),
                  pl.BlockSpec((1, _D), lambda i: (0, 0))],
        out_specs=pl.BlockSpec((_ROWBLK, _D), lambda i: (i, 0)),
        out_shape=jax.ShapeDtypeStruct((_NP, _D), jnp.float32),
    )(keep, p0, p1, wcp, bvec, bcp)


def kernel(x, edge_index, edge_weight, W, b, Wc, bc):
    pad = _EP - _E
    src1 = jnp.concatenate([edge_index[0], jnp.zeros((pad,), jnp.int32)])
    dst1 = jnp.concatenate([edge_index[1], jnp.zeros((pad,), jnp.int32)])
    w1 = jnp.concatenate([edge_weight, jnp.zeros((pad,), jnp.float32)])
    xp = jnp.pad(x, ((0, _NP - _N), (0, 0)))
    nclass = Wc.shape[1]

    t0, t1 = _tc_first(xp, jnp.concatenate([W[0, 0], W[0, 1]], axis=1))
    for l in range(W.shape[0]):
        keep, upper = _spmm_hop1(src1, dst1, w1, t0, t1)
        parts = _spmm_hop2(src1, dst1, w1, upper)
        p0, p1 = parts[0], parts[1]
        bvec = jnp.concatenate([b[l, 0], b[l, 1]])[None, :]
        if l + 1 < W.shape[0]:
            wcat = jnp.concatenate([W[l + 1, 0], W[l + 1, 1]], axis=1)
            t0, t1 = _tc_layer(keep, p0, p1, wcat, bvec)
        else:
            wcp = jnp.pad(Wc, ((0, 0), (0, _D - nclass)))
            bcp = jnp.pad(bc, (0, _D - nclass))[None, :]
            out = _tc_final(keep, p0, p1, wcp, bvec, bcp)
            return out[:_N, :nclass]
